# baseline (speedup 1.0000x reference)
import functools

import jax
import jax.numpy as jnp
from jax import lax
from jax.experimental import pallas as pl
from jax.experimental.pallas import tpu as pltpu
from jax.experimental.pallas import tpu_sc as plsc


BATCH = 4
SEQ = 8192
DIM = 1024

NUM_CORES = 2
NUM_SUBCORES = 16
NUM_WORKERS = NUM_CORES * NUM_SUBCORES

ROWS_PER_W = SEQ // NUM_WORKERS
CHUNK_ROWS = 16
NCHUNK = ROWS_PER_W // CHUNK_ROWS
CHUNK = CHUNK_ROWS * DIM
VREGS = CHUNK // 16

_mesh = plsc.VectorSubcoreMesh(core_axis_name="c", subcore_axis_name="s")


@functools.partial(
    pl.kernel,
    mesh=_mesh,
    out_type=jax.ShapeDtypeStruct((BATCH * SEQ * DIM,), jnp.float32),
    scratch_types=[
        pltpu.VMEM((CHUNK,), jnp.float32),
        pltpu.VMEM((CHUNK,), jnp.float32),
        pltpu.VMEM((CHUNK,), jnp.float32),
        pltpu.VMEM((CHUNK,), jnp.float32),
        pltpu.SemaphoreType.DMA,
        pltpu.SemaphoreType.DMA,
        pltpu.SemaphoreType.DMA,
        pltpu.SemaphoreType.DMA,
        pltpu.SemaphoreType.DMA,
        pltpu.SemaphoreType.DMA,
    ],
)
def _sc_add(x_hbm, e_hbm, o_hbm, e0, e1, x0, x1, se0, se1, sx0, sx1, so0, so1):
    wid = lax.axis_index("s") * NUM_CORES + lax.axis_index("c")
    base = wid * (ROWS_PER_W * DIM)

    ebufs, esems = (e0, e1), (se0, se1)
    xbufs, xsems = (x0, x1), (sx0, sx1)
    osems = (so0, so1)

    def start_e(c):
        off = base + c * CHUNK
        return pltpu.async_copy(
            e_hbm.at[pl.ds(off, CHUNK)], ebufs[c % 2], esems[c % 2])

    def start_x(k):
        c, b = k // 4, k % 4
        off = b * (SEQ * DIM) + base + c * CHUNK
        return pltpu.async_copy(
            x_hbm.at[pl.ds(off, CHUNK)], xbufs[k % 2], xsems[k % 2])

    def start_o(k):
        c, b = k // 4, k % 4
        off = b * (SEQ * DIM) + base + c * CHUNK
        return pltpu.async_copy(
            xbufs[k % 2], o_hbm.at[pl.ds(off, CHUNK)], osems[k % 2])

    def add_chunk(xbuf, ebuf):
        def body(i, carry):
            sl = pl.ds(i * 16, 16)
            plsc.addupdate(xbuf.at[sl], ebuf[sl])
            return carry
        lax.fori_loop(0, VREGS, body, 0, unroll=8)

    nsteps = NCHUNK * BATCH
    e_pending = {0: start_e(0), 1: start_e(1)}
    x_pending = {0: start_x(0)}
    o_pending = {}

    for k in range(nsteps):
        c, b = k // 4, k % 4
        if k + 1 < nsteps:
            if k - 1 in o_pending:
                o_pending.pop(k - 1).wait()
            x_pending[k + 1] = start_x(k + 1)
        if b == 0:
            e_pending.pop(c).wait()
        x_pending.pop(k).wait()
        add_chunk(xbufs[k % 2], ebufs[c % 2])
        o_pending[k] = start_o(k)
        if b == 3 and c + 2 < NCHUNK:
            e_pending[c + 2] = start_e(c + 2)

    for k in sorted(o_pending):
        o_pending.pop(k).wait()


def kernel(inputs, embeddings):
    batch, seq, dim = inputs.shape
    pos = embeddings[:seq]
    out_flat = _sc_add(inputs.reshape(-1), pos.reshape(-1))
    return out_flat.reshape(batch, seq, dim)


# speedup vs baseline: 1.0076x; 1.0076x over previous
import functools

import jax
import jax.numpy as jnp
from jax import lax
from jax.experimental import pallas as pl
from jax.experimental.pallas import tpu as pltpu
from jax.experimental.pallas import tpu_sc as plsc


BATCH = 4
SEQ = 8192
DIM = 1024

NUM_CORES = 2
NUM_SUBCORES = 16
NUM_WORKERS = NUM_CORES * NUM_SUBCORES

ROWS_PER_W = SEQ // NUM_WORKERS
CHUNK_ROWS = 16
NCHUNK = ROWS_PER_W // CHUNK_ROWS
CHUNK = CHUNK_ROWS * DIM
VREGS = CHUNK // 16

_mesh = plsc.VectorSubcoreMesh(core_axis_name="c", subcore_axis_name="s")


RING = 4


@functools.partial(
    pl.kernel,
    mesh=_mesh,
    out_type=jax.ShapeDtypeStruct((BATCH * SEQ * DIM,), jnp.float32),
    scratch_types=(
        [pltpu.VMEM((CHUNK,), jnp.float32) for _ in range(2 + RING)]
        + [pltpu.SemaphoreType.DMA for _ in range(2 + 2 * RING)]
    ),
)
def _sc_add(x_hbm, e_hbm, o_hbm, *rest):
    bufs, sems = rest[: 2 + RING], rest[2 + RING:]
    ebufs, xbufs = bufs[:2], bufs[2:]
    esems, xsems, osems = sems[:2], sems[2: 2 + RING], sems[2 + RING:]

    wid = lax.axis_index("s") * NUM_CORES + lax.axis_index("c")
    base = wid * (ROWS_PER_W * DIM)

    def start_e(c):
        off = base + c * CHUNK
        return pltpu.async_copy(
            e_hbm.at[pl.ds(off, CHUNK)], ebufs[c % 2], esems[c % 2])

    def start_x(k):
        c, b = k // 4, k % 4
        off = b * (SEQ * DIM) + base + c * CHUNK
        return pltpu.async_copy(
            x_hbm.at[pl.ds(off, CHUNK)], xbufs[k % RING], xsems[k % RING])

    def start_o(k):
        c, b = k // 4, k % 4
        off = b * (SEQ * DIM) + base + c * CHUNK
        return pltpu.async_copy(
            xbufs[k % RING], o_hbm.at[pl.ds(off, CHUNK)], osems[k % RING])

    def add_chunk(xbuf, ebuf):
        def body(i, carry):
            sl = pl.ds(i * 16, 16)
            plsc.addupdate(xbuf.at[sl], ebuf[sl])
            return carry
        lax.fori_loop(0, VREGS, body, 0, unroll=8)

    nsteps = NCHUNK * BATCH
    e_pending = {0: start_e(0), 1: start_e(1)}
    x_pending = {j: start_x(j) for j in range(RING - 1)}
    o_pending = {}

    for k in range(nsteps):
        c, b = k // 4, k % 4
        kn = k + RING - 1
        if kn < nsteps:
            if kn - RING in o_pending:
                o_pending.pop(kn - RING).wait()
            x_pending[kn] = start_x(kn)
        if b == 0:
            e_pending.pop(c).wait()
        x_pending.pop(k).wait()
        add_chunk(xbufs[k % RING], ebufs[c % 2])
        o_pending[k] = start_o(k)
        if b == 3 and c + 2 < NCHUNK:
            e_pending[c + 2] = start_e(c + 2)

    for k in sorted(o_pending):
        o_pending.pop(k).wait()


def kernel(inputs, embeddings):
    batch, seq, dim = inputs.shape
    pos = embeddings[:seq]
    out_flat = _sc_add(inputs.reshape(-1), pos.reshape(-1))
    return out_flat.reshape(batch, seq, dim)
